# baseline (speedup 1.0000x reference)
import functools

import jax
import jax.numpy as jnp
from jax import lax
from jax.experimental import pallas as pl
from jax.experimental.pallas import tpu as pltpu
from jax.experimental.pallas import tpu_sc as plsc

_NUM_CODES = 8192
_EMBED_DIM = 256
_N_TOKENS = 36864
_COMMIT = 0.1
_BLK = 256

_NC = 2
_NS = 16
_NW = _NC * _NS
_CHUNK = 128
_SCL = 16


def _tc_body(num_codes, n_tokens, dim, nblk,
             scale_ref, z_ref, emb_ref,
             idx_ref, a_ref, nsqn_ref, ppl_ref, loss_ref,
             embn_ref, psum_ref, iota_ref, acc_ref):
    i = pl.program_id(0)

    @pl.when(i == 0)
    def _init():
        e = emb_ref[...]
        nsq = jnp.sum(e * e, axis=1, keepdims=True)
        n = jnp.maximum(jnp.sqrt(nsq), 1e-12)
        embn_ref[...] = e / n
        nsqn_ref[...] = jnp.concatenate([nsq, n], axis=1)
        psum_ref[...] = jnp.zeros_like(psum_ref)
        iota_ref[...] = lax.broadcasted_iota(jnp.int32, iota_ref.shape, 1)
        acc_ref[0] = 0.0

    scale = scale_ref[0]
    z = z_ref[...]
    znsq = jnp.sum(z * z, axis=1, keepdims=True)
    zn = jnp.maximum(jnp.sqrt(znsq), 1e-12)
    zhat = z / zn
    logits = scale * lax.dot_general(
        zhat, embn_ref[...], (((1,), (1,)), ((), ())))

    m = jnp.max(logits, axis=1, keepdims=True)
    idx = jnp.min(jnp.where(logits == m, iota_ref[...], num_codes), axis=1,
                  keepdims=True)
    idx_ref[...] = idx.reshape(idx_ref.shape)
    a_ref[...] = (m * (1.0 / scale) * zn).reshape(a_ref.shape)
    acc_ref[0] += jnp.sum(znsq)

    p = jnp.exp(logits).astype(jnp.bfloat16)
    s = jnp.sum(p, axis=1, keepdims=True, dtype=jnp.float32)
    psum_ref[...] += lax.dot_general(
        (1.0 / s).astype(jnp.bfloat16), p, (((0,), (0,)), ((), ())),
        preferred_element_type=jnp.float32)

    @pl.when(i == nblk - 1)
    def _fini():
        avg = psum_ref[...] * (1.0 / n_tokens)
        ent = jnp.sum(avg * jnp.log(avg + 1e-10))
        ppl_ref[...] = jnp.broadcast_to(jnp.exp(-ent), (1, 1))
        loss_ref[...] = jnp.broadcast_to(
            acc_ref[0] * (_COMMIT / (n_tokens * dim)), (1, 1))


def _build_tc(n_tokens, num_codes, dim, blk, interpret=False):
    nblk = n_tokens // blk
    return pl.pallas_call(
        functools.partial(_tc_body, num_codes, n_tokens, dim, nblk),
        grid=(nblk,),
        in_specs=[
            pl.BlockSpec(memory_space=pltpu.SMEM),
            pl.BlockSpec((blk, dim), lambda i: (i, 0)),
            pl.BlockSpec((num_codes, dim), lambda i: (0, 0)),
        ],
        out_specs=[
            pl.BlockSpec((1, blk, 1), lambda i: (i, 0, 0)),
            pl.BlockSpec((1, blk, 1), lambda i: (i, 0, 0)),
            pl.BlockSpec((num_codes, 2), lambda i: (0, 0)),
            pl.BlockSpec((1, 1), lambda i: (0, 0)),
            pl.BlockSpec((1, 1), lambda i: (0, 0)),
        ],
        out_shape=[
            jax.ShapeDtypeStruct((nblk, blk, 1), jnp.int32),
            jax.ShapeDtypeStruct((nblk, blk, 1), jnp.float32),
            jax.ShapeDtypeStruct((num_codes, 2), jnp.float32),
            jax.ShapeDtypeStruct((1, 1), jnp.float32),
            jax.ShapeDtypeStruct((1, 1), jnp.float32),
        ],
        scratch_shapes=[
            pltpu.VMEM((num_codes, dim), jnp.float32),
            pltpu.VMEM((1, num_codes), jnp.float32),
            pltpu.VMEM((blk, num_codes), jnp.int32),
            pltpu.SMEM((1,), jnp.float32),
        ],
        interpret=interpret,
    )


def _build_sc(n_tokens, num_codes, dim):
    bpw = n_tokens // _NW
    nchunk = bpw // _CHUNK
    mesh = plsc.VectorSubcoreMesh(core_axis_name="c", subcore_axis_name="s")

    @functools.partial(
        pl.kernel, mesh=mesh,
        out_type=[
            jax.ShapeDtypeStruct((n_tokens, dim), jnp.float32),
            jax.ShapeDtypeStruct((_NW, _SCL), jnp.float32),
        ],
        scratch_types=[
            pltpu.VMEM((_CHUNK,), jnp.int32),
            pltpu.VMEM((_CHUNK,), jnp.float32),
            pltpu.VMEM((_CHUNK, dim), jnp.float32),
            pltpu.VMEM((_CHUNK,), jnp.float32),
            pltpu.VMEM((_CHUNK,), jnp.float32),
            pltpu.VMEM((_SCL,), jnp.float32),
            pltpu.SemaphoreType.DMA,
        ],
    )
    def sc_k(table_hbm, idx_hbm, a_hbm, nsq_hbm, n_hbm,
             out_hbm, part_hbm,
             idx_v, a_v, rows_v, nsqs_v, ns_v, acc_v, sem):
        wid = lax.axis_index("s") * _NC + lax.axis_index("c")
        base = wid * bpw
        acc = jnp.zeros((_SCL,), jnp.float32)
        for c in range(nchunk):
            off = base + c * _CHUNK
            pltpu.sync_copy(idx_hbm.at[pl.ds(off, _CHUNK)], idx_v)
            pltpu.sync_copy(a_hbm.at[pl.ds(off, _CHUNK)], a_v)
            pltpu.async_copy(table_hbm.at[idx_v], rows_v, sem).wait()
            pltpu.async_copy(nsq_hbm.at[idx_v], nsqs_v, sem).wait()
            pltpu.async_copy(n_hbm.at[idx_v], ns_v, sem).wait()
            pltpu.sync_copy(rows_v, out_hbm.at[pl.ds(off, _CHUNK)])
            for g in range(_CHUNK // _SCL):
                sl = pl.ds(g * _SCL, _SCL)
                acc = acc + (nsqs_v[sl] - 2.0 * a_v[sl] * ns_v[sl])
        acc_v[...] = acc
        pltpu.sync_copy(acc_v, part_hbm.at[wid])

    return sc_k


def kernel(z_e, embeddings, logit_scale):
    idx3, a3, nsqn, ppl, loss_tc = _build_tc(
        _N_TOKENS, _NUM_CODES, _EMBED_DIM, _BLK)(logit_scale, z_e, embeddings)
    indices = idx3.reshape(_N_TOKENS)
    a = a3.reshape(_N_TOKENS)
    nsq = nsqn[:, 0]
    n = nsqn[:, 1]
    z_q, parts = _build_sc(_N_TOKENS, _NUM_CODES, _EMBED_DIM)(
        embeddings, indices, a, nsq, n)
    loss = loss_tc[0, 0] + jnp.sum(parts) * (_COMMIT / (_N_TOKENS * _EMBED_DIM))
    return z_q, indices, ppl[0, 0], loss


# speedup vs baseline: 1.0498x; 1.0498x over previous
import functools

import jax
import jax.numpy as jnp
from jax import lax
from jax.experimental import pallas as pl
from jax.experimental.pallas import tpu as pltpu
from jax.experimental.pallas import tpu_sc as plsc

_NUM_CODES = 8192
_EMBED_DIM = 256
_N_TOKENS = 36864
_COMMIT = 0.1
_BLK = 256

_NC = 2
_NS = 16
_NW = _NC * _NS
_CHUNK = 128
_SCL = 16


def _tc_body(num_codes, n_tokens, dim, nblk,
             scale_ref, z_ref, emb_ref,
             idx_ref, a_ref, nsqn_ref, ppl_ref, loss_ref,
             embn_ref, psum_ref, iota_ref, acc_ref):
    i = pl.program_id(0)

    @pl.when(i == 0)
    def _init():
        e = emb_ref[...]
        nsq = jnp.sum(e * e, axis=1, keepdims=True)
        n = jnp.maximum(jnp.sqrt(nsq), 1e-12)
        embn_ref[...] = e / n
        nsqn_ref[...] = jnp.concatenate([nsq, n], axis=1)
        psum_ref[...] = jnp.zeros_like(psum_ref)
        iota_ref[...] = lax.broadcasted_iota(jnp.int32, iota_ref.shape, 1)
        acc_ref[0] = 0.0

    scale = scale_ref[0]
    z = z_ref[...]
    znsq = jnp.sum(z * z, axis=1, keepdims=True)
    zn = jnp.maximum(jnp.sqrt(znsq), 1e-12)
    zhat = z / zn
    logits = scale * lax.dot_general(
        zhat, embn_ref[...], (((1,), (1,)), ((), ())))

    m = jnp.max(logits, axis=1, keepdims=True)
    code_iota = lax.broadcasted_iota(jnp.int32, logits.shape, 1)
    idx = jnp.min(jnp.where(logits == m, code_iota, num_codes), axis=1,
                  keepdims=True)
    idx_ref[...] = idx.reshape(idx_ref.shape)
    a_ref[...] = (m * (1.0 / scale) * zn).reshape(a_ref.shape)
    acc_ref[0] += jnp.sum(znsq)

    p = jnp.exp(logits)
    s = jnp.sum(p, axis=1, keepdims=True)
    psum_ref[...] += lax.dot_general(
        1.0 / s, p, (((0,), (0,)), ((), ())))

    @pl.when(i == nblk - 1)
    def _fini():
        avg = psum_ref[...] * (1.0 / n_tokens)
        ent = jnp.sum(avg * jnp.log(avg + 1e-10))
        ppl_ref[...] = jnp.broadcast_to(jnp.exp(-ent), (1, 1))
        loss_ref[...] = jnp.broadcast_to(
            acc_ref[0] * (_COMMIT / (n_tokens * dim)), (1, 1))


def _build_tc(n_tokens, num_codes, dim, blk, interpret=False):
    nblk = n_tokens // blk
    return pl.pallas_call(
        functools.partial(_tc_body, num_codes, n_tokens, dim, nblk),
        grid=(nblk,),
        in_specs=[
            pl.BlockSpec(memory_space=pltpu.SMEM),
            pl.BlockSpec((blk, dim), lambda i: (i, 0)),
            pl.BlockSpec((num_codes, dim), lambda i: (0, 0)),
        ],
        out_specs=[
            pl.BlockSpec((1, blk, 1), lambda i: (i, 0, 0)),
            pl.BlockSpec((1, blk, 1), lambda i: (i, 0, 0)),
            pl.BlockSpec((num_codes, 2), lambda i: (0, 0)),
            pl.BlockSpec((1, 1), lambda i: (0, 0)),
            pl.BlockSpec((1, 1), lambda i: (0, 0)),
        ],
        out_shape=[
            jax.ShapeDtypeStruct((nblk, blk, 1), jnp.int32),
            jax.ShapeDtypeStruct((nblk, blk, 1), jnp.float32),
            jax.ShapeDtypeStruct((num_codes, 2), jnp.float32),
            jax.ShapeDtypeStruct((1, 1), jnp.float32),
            jax.ShapeDtypeStruct((1, 1), jnp.float32),
        ],
        scratch_shapes=[
            pltpu.VMEM((num_codes, dim), jnp.float32),
            pltpu.VMEM((1, num_codes), jnp.float32),
            pltpu.VMEM((blk, num_codes), jnp.int32),
            pltpu.SMEM((1,), jnp.float32),
        ],
        interpret=interpret,
    )


def _build_sc(n_tokens, num_codes, dim):
    bpw = n_tokens // _NW
    nchunk = bpw // _CHUNK
    mesh = plsc.VectorSubcoreMesh(core_axis_name="c", subcore_axis_name="s")

    @functools.partial(
        pl.kernel, mesh=mesh,
        out_type=[
            jax.ShapeDtypeStruct((n_tokens, dim), jnp.float32),
            jax.ShapeDtypeStruct((_NW, _SCL), jnp.float32),
        ],
        scratch_types=[
            pltpu.VMEM((_CHUNK,), jnp.int32),
            pltpu.VMEM((_CHUNK,), jnp.float32),
            pltpu.VMEM((_CHUNK, dim), jnp.float32),
            pltpu.VMEM((_CHUNK,), jnp.float32),
            pltpu.VMEM((_CHUNK,), jnp.float32),
            pltpu.VMEM((_SCL,), jnp.float32),
            pltpu.SemaphoreType.DMA,
        ],
    )
    def sc_k(table_hbm, idx_hbm, a_hbm, nsq_hbm, n_hbm,
             out_hbm, part_hbm,
             idx_v, a_v, rows_v, nsqs_v, ns_v, acc_v, sem):
        wid = lax.axis_index("s") * _NC + lax.axis_index("c")
        base = wid * bpw
        acc = jnp.zeros((_SCL,), jnp.float32)
        for c in range(nchunk):
            off = base + c * _CHUNK
            pltpu.sync_copy(idx_hbm.at[pl.ds(off, _CHUNK)], idx_v)
            pltpu.sync_copy(a_hbm.at[pl.ds(off, _CHUNK)], a_v)
            pltpu.async_copy(table_hbm.at[idx_v], rows_v, sem).wait()
            pltpu.async_copy(nsq_hbm.at[idx_v], nsqs_v, sem).wait()
            pltpu.async_copy(n_hbm.at[idx_v], ns_v, sem).wait()
            pltpu.sync_copy(rows_v, out_hbm.at[pl.ds(off, _CHUNK)])
            for g in range(_CHUNK // _SCL):
                sl = pl.ds(g * _SCL, _SCL)
                acc = acc + (nsqs_v[sl] - 2.0 * a_v[sl] * ns_v[sl])
        acc_v[...] = acc
        pltpu.sync_copy(acc_v, part_hbm.at[wid])

    return sc_k


def kernel(z_e, embeddings, logit_scale):
    idx3, a3, nsqn, ppl, loss_tc = _build_tc(
        _N_TOKENS, _NUM_CODES, _EMBED_DIM, _BLK)(logit_scale, z_e, embeddings)
    indices = idx3.reshape(_N_TOKENS)
    a = a3.reshape(_N_TOKENS)
    nsq = nsqn[:, 0]
    n = nsqn[:, 1]
    z_q, parts = _build_sc(_N_TOKENS, _NUM_CODES, _EMBED_DIM)(
        embeddings, indices, a, nsq, n)
    loss = loss_tc[0, 0] + jnp.sum(parts) * (_COMMIT / (_N_TOKENS * _EMBED_DIM))
    return z_q, indices, ppl[0, 0], loss


# speedup vs baseline: 1.0625x; 1.0121x over previous
import functools

import jax
import jax.numpy as jnp
from jax import lax
from jax.experimental import pallas as pl
from jax.experimental.pallas import tpu as pltpu
from jax.experimental.pallas import tpu_sc as plsc

_NUM_CODES = 8192
_EMBED_DIM = 256
_N_TOKENS = 36864
_COMMIT = 0.1
_BLK = 384

_NC = 2
_NS = 16
_NW = _NC * _NS
_CHUNK = 128
_SCL = 16


def _tc_body(num_codes, n_tokens, dim, nblk,
             scale_ref, z_ref, emb_ref,
             idx_ref, a_ref, nsqn_ref, ppl_ref, loss_ref,
             embn_ref, psum_ref, iota_ref, acc_ref):
    i = pl.program_id(0)

    @pl.when(i == 0)
    def _init():
        e = emb_ref[...]
        nsq = jnp.sum(e * e, axis=1, keepdims=True)
        n = jnp.maximum(jnp.sqrt(nsq), 1e-12)
        embn_ref[...] = e / n
        nsqn_ref[...] = jnp.concatenate([nsq, n], axis=1)
        psum_ref[...] = jnp.zeros_like(psum_ref)
        iota_ref[...] = lax.broadcasted_iota(jnp.int32, iota_ref.shape, 1)
        acc_ref[0] = 0.0

    scale = scale_ref[0]
    z = z_ref[...]
    znsq = jnp.sum(z * z, axis=1, keepdims=True)
    zn = jnp.maximum(jnp.sqrt(znsq), 1e-12)
    zhat = z / zn
    logits = scale * lax.dot_general(
        zhat, embn_ref[...], (((1,), (1,)), ((), ())))

    m = jnp.max(logits, axis=1, keepdims=True)
    idx = jnp.min(jnp.where(logits == m, iota_ref[...], num_codes), axis=1,
                  keepdims=True)
    idx_ref[...] = idx.reshape(idx_ref.shape)
    a_ref[...] = (m * (1.0 / scale) * zn).reshape(a_ref.shape)
    acc_ref[0] += jnp.sum(znsq)

    p = jnp.exp(logits)
    s = jnp.sum(p, axis=1, keepdims=True)
    psum_ref[...] += lax.dot_general(
        1.0 / s, p, (((0,), (0,)), ((), ())))

    @pl.when(i == nblk - 1)
    def _fini():
        avg = psum_ref[...] * (1.0 / n_tokens)
        ent = jnp.sum(avg * jnp.log(avg + 1e-10))
        ppl_ref[...] = jnp.broadcast_to(jnp.exp(-ent), (1, 1))
        loss_ref[...] = jnp.broadcast_to(
            acc_ref[0] * (_COMMIT / (n_tokens * dim)), (1, 1))


def _build_tc(n_tokens, num_codes, dim, blk, interpret=False):
    nblk = n_tokens // blk
    return pl.pallas_call(
        functools.partial(_tc_body, num_codes, n_tokens, dim, nblk),
        grid=(nblk,),
        in_specs=[
            pl.BlockSpec(memory_space=pltpu.SMEM),
            pl.BlockSpec((blk, dim), lambda i: (i, 0)),
            pl.BlockSpec((num_codes, dim), lambda i: (0, 0)),
        ],
        out_specs=[
            pl.BlockSpec((1, blk, 1), lambda i: (i, 0, 0)),
            pl.BlockSpec((1, blk, 1), lambda i: (i, 0, 0)),
            pl.BlockSpec((num_codes, 2), lambda i: (0, 0)),
            pl.BlockSpec((1, 1), lambda i: (0, 0)),
            pl.BlockSpec((1, 1), lambda i: (0, 0)),
        ],
        out_shape=[
            jax.ShapeDtypeStruct((nblk, blk, 1), jnp.int32),
            jax.ShapeDtypeStruct((nblk, blk, 1), jnp.float32),
            jax.ShapeDtypeStruct((num_codes, 2), jnp.float32),
            jax.ShapeDtypeStruct((1, 1), jnp.float32),
            jax.ShapeDtypeStruct((1, 1), jnp.float32),
        ],
        scratch_shapes=[
            pltpu.VMEM((num_codes, dim), jnp.float32),
            pltpu.VMEM((1, num_codes), jnp.float32),
            pltpu.VMEM((blk, num_codes), jnp.int32),
            pltpu.SMEM((1,), jnp.float32),
        ],
        interpret=interpret,
    )


def _build_sc(n_tokens, num_codes, dim):
    bpw = n_tokens // _NW
    nchunk = bpw // _CHUNK
    mesh = plsc.VectorSubcoreMesh(core_axis_name="c", subcore_axis_name="s")

    @functools.partial(
        pl.kernel, mesh=mesh,
        out_type=[
            jax.ShapeDtypeStruct((n_tokens, dim), jnp.float32),
            jax.ShapeDtypeStruct((_NW, _SCL), jnp.float32),
        ],
        scratch_types=[
            pltpu.VMEM((_CHUNK,), jnp.int32),
            pltpu.VMEM((_CHUNK,), jnp.float32),
            pltpu.VMEM((_CHUNK, dim), jnp.float32),
            pltpu.VMEM((_CHUNK,), jnp.float32),
            pltpu.VMEM((_CHUNK,), jnp.float32),
            pltpu.VMEM((_SCL,), jnp.float32),
            pltpu.SemaphoreType.DMA,
        ],
    )
    def sc_k(table_hbm, idx_hbm, a_hbm, nsq_hbm, n_hbm,
             out_hbm, part_hbm,
             idx_v, a_v, rows_v, nsqs_v, ns_v, acc_v, sem):
        wid = lax.axis_index("s") * _NC + lax.axis_index("c")
        base = wid * bpw
        acc = jnp.zeros((_SCL,), jnp.float32)
        for c in range(nchunk):
            off = base + c * _CHUNK
            pltpu.sync_copy(idx_hbm.at[pl.ds(off, _CHUNK)], idx_v)
            pltpu.sync_copy(a_hbm.at[pl.ds(off, _CHUNK)], a_v)
            pltpu.async_copy(table_hbm.at[idx_v], rows_v, sem).wait()
            pltpu.async_copy(nsq_hbm.at[idx_v], nsqs_v, sem).wait()
            pltpu.async_copy(n_hbm.at[idx_v], ns_v, sem).wait()
            pltpu.sync_copy(rows_v, out_hbm.at[pl.ds(off, _CHUNK)])
            for g in range(_CHUNK // _SCL):
                sl = pl.ds(g * _SCL, _SCL)
                acc = acc + (nsqs_v[sl] - 2.0 * a_v[sl] * ns_v[sl])
        acc_v[...] = acc
        pltpu.sync_copy(acc_v, part_hbm.at[wid])

    return sc_k


def kernel(z_e, embeddings, logit_scale):
    idx3, a3, nsqn, ppl, loss_tc = _build_tc(
        _N_TOKENS, _NUM_CODES, _EMBED_DIM, _BLK)(logit_scale, z_e, embeddings)
    indices = idx3.reshape(_N_TOKENS)
    a = a3.reshape(_N_TOKENS)
    nsq = nsqn[:, 0]
    n = nsqn[:, 1]
    z_q, parts = _build_sc(_N_TOKENS, _NUM_CODES, _EMBED_DIM)(
        embeddings, indices, a, nsq, n)
    loss = loss_tc[0, 0] + jnp.sum(parts) * (_COMMIT / (_N_TOKENS * _EMBED_DIM))
    return z_q, indices, ppl[0, 0], loss


# speedup vs baseline: 1.1002x; 1.0354x over previous
import functools

import jax
import jax.numpy as jnp
from jax import lax
from jax.experimental import pallas as pl
from jax.experimental.pallas import tpu as pltpu
from jax.experimental.pallas import tpu_sc as plsc

_NUM_CODES = 8192
_EMBED_DIM = 256
_N_TOKENS = 36864
_COMMIT = 0.1
_BLK = 384

_NC = 2
_NS = 16
_NW = _NC * _NS
_CHUNK = 128
_SCL = 16


def _tc_body(num_codes, n_tokens, dim, nblk,
             scale_ref, z_ref, emb_ref,
             idx_ref, a_ref, nsqn_ref, ppl_ref, loss_ref,
             embn_ref, psum_ref, iota_ref, acc_ref):
    i = pl.program_id(0)

    @pl.when(i == 0)
    def _init():
        e = emb_ref[...]
        nsq = jnp.sum(e * e, axis=1, keepdims=True)
        n = jnp.maximum(jnp.sqrt(nsq), 1e-12)
        embn_ref[...] = e / n
        nsqn_ref[...] = jnp.concatenate([nsq, n], axis=1)
        psum_ref[...] = jnp.zeros_like(psum_ref)
        iota_ref[...] = lax.broadcasted_iota(jnp.int32, iota_ref.shape, 1)
        acc_ref[0] = 0.0

    scale = scale_ref[0]
    z = z_ref[...]
    znsq = jnp.sum(z * z, axis=1, keepdims=True)
    zn = jnp.maximum(jnp.sqrt(znsq), 1e-12)
    zhat = z / zn
    logits = scale * lax.dot_general(
        zhat, embn_ref[...], (((1,), (1,)), ((), ())))

    m = jnp.max(logits, axis=1, keepdims=True)
    idx = jnp.min(jnp.where(logits == m, iota_ref[...], num_codes), axis=1,
                  keepdims=True)
    idx_ref[...] = idx.reshape(idx_ref.shape)
    a_ref[...] = (m * (1.0 / scale) * zn).reshape(a_ref.shape)
    acc_ref[0] += jnp.sum(znsq)

    p = jnp.exp(logits)
    s = jnp.sum(p, axis=1, keepdims=True)
    psum_ref[...] += lax.dot_general(
        1.0 / s, p, (((0,), (0,)), ((), ())))

    @pl.when(i == nblk - 1)
    def _fini():
        avg = psum_ref[...] * (1.0 / n_tokens)
        ent = jnp.sum(avg * jnp.log(avg + 1e-10))
        ppl_ref[...] = jnp.broadcast_to(jnp.exp(-ent), (1, 1))
        loss_ref[...] = jnp.broadcast_to(
            acc_ref[0] * (_COMMIT / (n_tokens * dim)), (1, 1))


def _build_tc(n_tokens, num_codes, dim, blk, interpret=False):
    nblk = n_tokens // blk
    return pl.pallas_call(
        functools.partial(_tc_body, num_codes, n_tokens, dim, nblk),
        grid=(nblk,),
        in_specs=[
            pl.BlockSpec(memory_space=pltpu.SMEM),
            pl.BlockSpec((blk, dim), lambda i: (i, 0)),
            pl.BlockSpec((num_codes, dim), lambda i: (0, 0)),
        ],
        out_specs=[
            pl.BlockSpec((1, blk, 1), lambda i: (i, 0, 0)),
            pl.BlockSpec((1, blk, 1), lambda i: (i, 0, 0)),
            pl.BlockSpec((num_codes, 2), lambda i: (0, 0)),
            pl.BlockSpec((1, 1), lambda i: (0, 0)),
            pl.BlockSpec((1, 1), lambda i: (0, 0)),
        ],
        out_shape=[
            jax.ShapeDtypeStruct((nblk, blk, 1), jnp.int32),
            jax.ShapeDtypeStruct((nblk, blk, 1), jnp.float32),
            jax.ShapeDtypeStruct((num_codes, 2), jnp.float32),
            jax.ShapeDtypeStruct((1, 1), jnp.float32),
            jax.ShapeDtypeStruct((1, 1), jnp.float32),
        ],
        scratch_shapes=[
            pltpu.VMEM((num_codes, dim), jnp.float32),
            pltpu.VMEM((1, num_codes), jnp.float32),
            pltpu.VMEM((blk, num_codes), jnp.int32),
            pltpu.SMEM((1,), jnp.float32),
        ],
        interpret=interpret,
    )


def _build_sc(n_tokens, num_codes, dim):
    bpw = n_tokens // _NW
    nchunk = bpw // _CHUNK
    mesh = plsc.VectorSubcoreMesh(core_axis_name="c", subcore_axis_name="s")

    @functools.partial(
        pl.kernel, mesh=mesh,
        out_type=[
            jax.ShapeDtypeStruct((n_tokens, dim), jnp.float32),
            jax.ShapeDtypeStruct((_NW, _SCL), jnp.float32),
        ],
        scratch_types=[
            pltpu.VMEM((bpw,), jnp.int32),
            pltpu.VMEM((bpw,), jnp.float32),
            pltpu.VMEM((2, _CHUNK, dim), jnp.float32),
            pltpu.VMEM((2, _CHUNK), jnp.float32),
            pltpu.VMEM((2, _CHUNK), jnp.float32),
            pltpu.VMEM((_SCL,), jnp.float32),
            pltpu.SemaphoreType.DMA((2,)),
            pltpu.SemaphoreType.DMA((2,)),
        ],
    )
    def sc_k(table_hbm, idx_hbm, a_hbm, nsq_hbm, n_hbm,
             out_hbm, part_hbm,
             idx_v, a_v, rows_v, nsqs_v, ns_v, acc_v, gsem, osem):
        wid = lax.axis_index("s") * _NC + lax.axis_index("c")
        base = wid * bpw
        pltpu.sync_copy(idx_hbm.at[pl.ds(base, bpw)], idx_v)
        pltpu.sync_copy(a_hbm.at[pl.ds(base, bpw)], a_v)

        def fire(c):
            b = c % 2
            ids = idx_v.at[pl.ds(c * _CHUNK, _CHUNK)]
            return (
                pltpu.async_copy(table_hbm.at[ids], rows_v.at[b], gsem.at[b]),
                pltpu.async_copy(nsq_hbm.at[ids], nsqs_v.at[b], gsem.at[b]),
                pltpu.async_copy(n_hbm.at[ids], ns_v.at[b], gsem.at[b]),
            )

        acc = jnp.zeros((_SCL,), jnp.float32)
        pending = fire(0)
        outs = [None, None]
        for c in range(nchunk):
            b = c % 2
            if c + 1 < nchunk:
                if outs[1 - b] is not None:
                    outs[1 - b].wait()
                    outs[1 - b] = None
                nxt = fire(c + 1)
            for d in pending:
                d.wait()
            outs[b] = pltpu.async_copy(
                rows_v.at[b], out_hbm.at[pl.ds(base + c * _CHUNK, _CHUNK)],
                osem.at[b])
            for g in range(_CHUNK // _SCL):
                sl = pl.ds(g * _SCL, _SCL)
                asl = pl.ds(c * _CHUNK + g * _SCL, _SCL)
                acc = acc + (nsqs_v[b, sl] - 2.0 * a_v[asl] * ns_v[b, sl])
            if c + 1 < nchunk:
                pending = nxt
        for o in outs:
            if o is not None:
                o.wait()
        acc_v[...] = acc
        pltpu.sync_copy(acc_v, part_hbm.at[wid])

    return sc_k


def kernel(z_e, embeddings, logit_scale):
    idx3, a3, nsqn, ppl, loss_tc = _build_tc(
        _N_TOKENS, _NUM_CODES, _EMBED_DIM, _BLK)(logit_scale, z_e, embeddings)
    indices = idx3.reshape(_N_TOKENS)
    a = a3.reshape(_N_TOKENS)
    nsq = nsqn[:, 0]
    n = nsqn[:, 1]
    z_q, parts = _build_sc(_N_TOKENS, _NUM_CODES, _EMBED_DIM)(
        embeddings, indices, a, nsq, n)
    loss = loss_tc[0, 0] + jnp.sum(parts) * (_COMMIT / (_N_TOKENS * _EMBED_DIM))
    return z_q, indices, ppl[0, 0], loss
